# baseline (device time: 74005 ns/iter reference)
import jax
import jax.numpy as jnp
from jax import lax
from jax.experimental import pallas as pl
from jax.experimental.pallas import tpu as pltpu

C = 8
FT = 1024


def kernel(x, assign, W1, W2):
    t, d = x.shape
    e_per, _, f = W1.shape
    assign2d = assign.reshape(t, 1)
    CH = t // C

    def body(x_ref, a_ref, w1_ref, w2_ref, out_ref,
             xsend, xin, arem, pout, pin, sin, send_sems, recv_sems):
        my_x = lax.axis_index("x")
        my_y = lax.axis_index("y")
        xpeer = (1 - my_x, my_y)
        ypeer = (my_x, 1 - my_y)
        is_home = my_x == my_y

        S0, SA, S1, S2 = 0, C, C + 1, 2 * C + 1

        barrier_sem = pltpu.get_barrier_semaphore()
        for nbr in (xpeer, ypeer):
            pl.semaphore_signal(barrier_sem, inc=1, device_id=nbr,
                                device_id_type=pl.DeviceIdType.MESH)
        pl.semaphore_wait(barrier_sem, 2)

        def ffn_chunk(xs, asg):
            xm = {}
            for el in range(e_per):
                e_gl = my_x * e_per + el
                xm[el] = jnp.where(asg == e_gl, xs, jnp.bfloat16(0))
            acc = jnp.zeros((CH, d), jnp.float32)
            for el in range(e_per):
                for fc in range(0, f, FT):
                    w1b = w1_ref[el, :, fc:fc + FT].astype(jnp.bfloat16)
                    w2b = w2_ref[el, fc:fc + FT, :].astype(jnp.bfloat16)
                    h = jnp.maximum(
                        jnp.dot(xm[el], w1b,
                                preferred_element_type=jnp.float32),
                        0.0).astype(jnp.bfloat16)
                    acc = acc + jnp.dot(h, w2b,
                                        preferred_element_type=jnp.float32)
            return acc

        def mk(src, dst, i, dev):
            return pltpu.make_async_remote_copy(
                src_ref=src, dst_ref=dst,
                send_sem=send_sems.at[i], recv_sem=recv_sems.at[i],
                device_id=dev, device_id_type=pl.DeviceIdType.MESH)

        @pl.when(is_home)
        def _home():
            r0a = mk(a_ref, arem, SA, xpeer)
            r0a.start()
            r0x = []
            for c in range(C):
                rs = slice(c * CH, (c + 1) * CH)
                xsend[rs] = x_ref[rs].astype(jnp.bfloat16)
                r = mk(xsend.at[pl.ds(c * CH, CH)],
                       xin.at[pl.ds(c * CH, CH)], S0 + c, xpeer)
                r.start()
                r0x.append(r)

            for c in range(C):
                rs = slice(c * CH, (c + 1) * CH)
                out_ref[rs] = ffn_chunk(xsend[rs], a_ref[rs])

            r2 = []
            for c in range(C):
                rs = slice(c * CH, (c + 1) * CH)
                rds = pl.ds(c * CH, CH)
                r1 = mk(pout.at[rds], pin.at[rds], S1 + c, xpeer)
                r1.wait_recv()
                out_ref[rs] = out_ref[rs] + pin[rs].astype(jnp.float32)
                pout[rs] = out_ref[rs].astype(jnp.bfloat16)
                r = mk(pout.at[rds], sin.at[rds], S2 + c, ypeer)
                r.start()
                r2.append(r)
            for r in r2:
                r.wait_send()
            for r in r0x:
                r.wait_send()
            r0a.wait_send()

        @pl.when(jnp.logical_not(is_home))
        def _away():
            r0a = mk(a_ref, arem, SA, xpeer)
            r0a.wait_recv()
            r1 = []
            for c in range(C):
                rs = slice(c * CH, (c + 1) * CH)
                rds = pl.ds(c * CH, CH)
                r0 = mk(xsend.at[rds], xin.at[rds], S0 + c, xpeer)
                r0.wait_recv()
                acc = ffn_chunk(xin[rs], arem[rs])
                pout[rs] = acc.astype(jnp.bfloat16)
                r = mk(pout.at[rds], pin.at[rds], S1 + c, xpeer)
                r.start()
                r1.append(r)
            for c in range(C):
                rs = slice(c * CH, (c + 1) * CH)
                rds = pl.ds(c * CH, CH)
                r2 = mk(pout.at[rds], sin.at[rds], S2 + c, ypeer)
                r2.wait_recv()
                out_ref[rs] = sin[rs].astype(jnp.float32)
            for r in r1:
                r.wait_send()

    return pl.pallas_call(
        body,
        out_shape=jax.ShapeDtypeStruct((t, d), jnp.float32),
        in_specs=[pl.BlockSpec(memory_space=pltpu.VMEM)] * 4,
        out_specs=pl.BlockSpec(memory_space=pltpu.VMEM),
        scratch_shapes=[
            pltpu.VMEM((t, d), jnp.bfloat16),
            pltpu.VMEM((t, d), jnp.bfloat16),
            pltpu.VMEM((t, 1), jnp.int32),
            pltpu.VMEM((t, d), jnp.bfloat16),
            pltpu.VMEM((t, d), jnp.bfloat16),
            pltpu.VMEM((t, d), jnp.bfloat16),
            pltpu.SemaphoreType.DMA((3 * C + 1,)),
            pltpu.SemaphoreType.DMA((3 * C + 1,)),
        ],
        compiler_params=pltpu.CompilerParams(
            collective_id=0, vmem_limit_bytes=100 * 1024 * 1024),
    )(x, assign2d, W1, W2)


# device time: 71628 ns/iter; 1.0332x vs baseline; 1.0332x over previous
import jax
import jax.numpy as jnp
from jax import lax
from jax.experimental import pallas as pl
from jax.experimental.pallas import tpu as pltpu

C = 4
FT = 2048


def kernel(x, assign, W1, W2):
    t, d = x.shape
    e_per, _, f = W1.shape
    assign2d = assign.reshape(t, 1)
    CH = t // C

    def body(x_ref, a_ref, w1_ref, w2_ref, out_ref,
             xsend, xin, arem, pout, pin, sin, send_sems, recv_sems):
        my_x = lax.axis_index("x")
        my_y = lax.axis_index("y")
        xpeer = (1 - my_x, my_y)
        ypeer = (my_x, 1 - my_y)
        is_home = my_x == my_y

        S0, SA, S1, S2 = 0, C, C + 1, 2 * C + 1

        barrier_sem = pltpu.get_barrier_semaphore()
        for nbr in (xpeer, ypeer):
            pl.semaphore_signal(barrier_sem, inc=1, device_id=nbr,
                                device_id_type=pl.DeviceIdType.MESH)
        pl.semaphore_wait(barrier_sem, 2)

        def ffn_chunk(xs, asg):
            xm = {}
            for el in range(e_per):
                e_gl = my_x * e_per + el
                xm[el] = jnp.where(asg == e_gl, xs, jnp.bfloat16(0))
            acc = jnp.zeros((CH, d), jnp.float32)
            for el in range(e_per):
                for fc in range(0, f, FT):
                    w1b = w1_ref[el, :, fc:fc + FT].astype(jnp.bfloat16)
                    w2b = w2_ref[el, fc:fc + FT, :].astype(jnp.bfloat16)
                    h = jnp.maximum(
                        jnp.dot(xm[el], w1b,
                                preferred_element_type=jnp.float32),
                        0.0).astype(jnp.bfloat16)
                    acc = acc + jnp.dot(h, w2b,
                                        preferred_element_type=jnp.float32)
            return acc

        def mk(src, dst, i, dev):
            return pltpu.make_async_remote_copy(
                src_ref=src, dst_ref=dst,
                send_sem=send_sems.at[i], recv_sem=recv_sems.at[i],
                device_id=dev, device_id_type=pl.DeviceIdType.MESH)

        @pl.when(is_home)
        def _home():
            r0a = mk(a_ref, arem, SA, xpeer)
            r0a.start()
            r0x = []
            for c in range(C):
                rs = slice(c * CH, (c + 1) * CH)
                xsend[rs] = x_ref[rs].astype(jnp.bfloat16)
                r = mk(xsend.at[pl.ds(c * CH, CH)],
                       xin.at[pl.ds(c * CH, CH)], S0 + c, xpeer)
                r.start()
                r0x.append(r)

            for c in range(C):
                rs = slice(c * CH, (c + 1) * CH)
                out_ref[rs] = ffn_chunk(xsend[rs], a_ref[rs])

            r2 = []
            for c in range(C):
                rs = slice(c * CH, (c + 1) * CH)
                rds = pl.ds(c * CH, CH)
                r1 = mk(pout.at[rds], pin.at[rds], S1 + c, xpeer)
                r1.wait_recv()
                out_ref[rs] = out_ref[rs] + pin[rs].astype(jnp.float32)
                pout[rs] = out_ref[rs].astype(jnp.bfloat16)
                r = mk(pout.at[rds], sin.at[rds], S2 + c, ypeer)
                r.start()
                r2.append(r)
            for r in r2:
                r.wait_send()
            for r in r0x:
                r.wait_send()
            r0a.wait_send()

        @pl.when(jnp.logical_not(is_home))
        def _away():
            r0a = mk(a_ref, arem, SA, xpeer)
            r0a.wait_recv()
            r1 = []
            for c in range(C):
                rs = slice(c * CH, (c + 1) * CH)
                rds = pl.ds(c * CH, CH)
                r0 = mk(xsend.at[rds], xin.at[rds], S0 + c, xpeer)
                r0.wait_recv()
                acc = ffn_chunk(xin[rs], arem[rs])
                pout[rs] = acc.astype(jnp.bfloat16)
                r = mk(pout.at[rds], pin.at[rds], S1 + c, xpeer)
                r.start()
                r1.append(r)
            for c in range(C):
                rs = slice(c * CH, (c + 1) * CH)
                rds = pl.ds(c * CH, CH)
                r2 = mk(pout.at[rds], sin.at[rds], S2 + c, ypeer)
                r2.wait_recv()
                out_ref[rs] = sin[rs].astype(jnp.float32)
            for r in r1:
                r.wait_send()

    return pl.pallas_call(
        body,
        out_shape=jax.ShapeDtypeStruct((t, d), jnp.float32),
        in_specs=[pl.BlockSpec(memory_space=pltpu.VMEM)] * 4,
        out_specs=pl.BlockSpec(memory_space=pltpu.VMEM),
        scratch_shapes=[
            pltpu.VMEM((t, d), jnp.bfloat16),
            pltpu.VMEM((t, d), jnp.bfloat16),
            pltpu.VMEM((t, 1), jnp.int32),
            pltpu.VMEM((t, d), jnp.bfloat16),
            pltpu.VMEM((t, d), jnp.bfloat16),
            pltpu.VMEM((t, d), jnp.bfloat16),
            pltpu.SemaphoreType.DMA((3 * C + 1,)),
            pltpu.SemaphoreType.DMA((3 * C + 1,)),
        ],
        compiler_params=pltpu.CompilerParams(
            collective_id=0, vmem_limit_bytes=100 * 1024 * 1024),
    )(x, assign2d, W1, W2)
